# Initial kernel scaffold; baseline (speedup 1.0000x reference)
import jax
import jax.numpy as jnp
from jax.experimental import pallas as pl


def kernel(x, token_table, pos_table):
    raise NotImplementedError("write your pallas kernel here")


# speedup vs baseline: 4.5832x; 4.5832x over previous
import functools

import jax
import jax.numpy as jnp
from jax import lax
from jax.experimental import pallas as pl
from jax.experimental.pallas import tpu as pltpu
from jax.experimental.pallas import tpu_sc as plsc

B = 4096
S = 200
D = 128
N = B * S
NW = 32
PER_W = N // NW
CHUNK = S
NCHUNK = PER_W // CHUNK


def _make_sc_kernel():
    mesh = plsc.VectorSubcoreMesh(core_axis_name="c", subcore_axis_name="s")

    @functools.partial(
        pl.kernel,
        mesh=mesh,
        out_type=jax.ShapeDtypeStruct((N, D), jnp.float32),
        scratch_types=[
            pltpu.VMEM((PER_W,), jnp.int32),
            pltpu.VMEM((S, D), jnp.float32),
            pltpu.VMEM((CHUNK, D), jnp.float32),
            pltpu.SemaphoreType.DMA,
        ],
    )
    def k(x_hbm, tok_hbm, pos_hbm, out_hbm, idx_v, pos_v, rows_v, sem):
        cid = lax.axis_index("c")
        sid = lax.axis_index("s")
        wid = sid * 2 + cid
        base = wid * PER_W

        pltpu.sync_copy(x_hbm.at[pl.ds(base, PER_W)], idx_v)
        pltpu.sync_copy(pos_hbm, pos_v)

        def chunk_body(c, carry):
            off = c * CHUNK
            cp1 = pltpu.async_copy(
                tok_hbm.at[idx_v.at[pl.ds(off, 128)]],
                rows_v.at[pl.ds(0, 128)], sem)
            cp2 = pltpu.async_copy(
                tok_hbm.at[idx_v.at[pl.ds(off + 128, 72)]],
                rows_v.at[pl.ds(128, 72)], sem)
            cp1.wait()
            cp2.wait()

            def row_body(i, carry2):
                for j in range(D // 16):
                    sl = pl.ds(j * 16, 16)
                    rows_v[i, sl] = rows_v[i, sl] + pos_v[i, sl]
                return carry2

            lax.fori_loop(0, CHUNK, row_body, 0)
            pltpu.sync_copy(rows_v, out_hbm.at[pl.ds(base + off, CHUNK)])
            return carry

        lax.fori_loop(0, NCHUNK, chunk_body, 0)

    return k


_sc_kernel = _make_sc_kernel()


def kernel(x, token_table, pos_table):
    xf = x.reshape(N).astype(jnp.int32)
    out = _sc_kernel(xf, token_table, pos_table)
    return out.reshape(B, S, D)


# speedup vs baseline: 9.0280x; 1.9698x over previous
import functools

import jax
import jax.numpy as jnp
from jax import lax
from jax.experimental import pallas as pl
from jax.experimental.pallas import tpu as pltpu
from jax.experimental.pallas import tpu_sc as plsc

B = 4096
S = 200
D = 128
N = B * S
NW = 32
PER_W = N // NW
CHUNK = S
NCHUNK = PER_W // CHUNK
NBUF = 3


def _make_sc_kernel():
    mesh = plsc.VectorSubcoreMesh(core_axis_name="c", subcore_axis_name="s")

    @functools.partial(
        pl.kernel,
        mesh=mesh,
        out_type=jax.ShapeDtypeStruct((N, D), jnp.float32),
        scratch_types=[
            pltpu.VMEM((PER_W,), jnp.int32),
            pltpu.VMEM((S, D), jnp.float32),
            pltpu.VMEM((NBUF, CHUNK, D), jnp.float32),
            pltpu.SemaphoreType.DMA,
            pltpu.SemaphoreType.DMA,
            pltpu.SemaphoreType.DMA,
            pltpu.SemaphoreType.DMA,
            pltpu.SemaphoreType.DMA,
            pltpu.SemaphoreType.DMA,
        ],
    )
    def k(x_hbm, tok_hbm, pos_hbm, out_hbm, idx_v, pos_v, rows_v,
          g0, g1, g2, o0, o1, o2):
        gsems = (g0, g1, g2)
        osems = (o0, o1, o2)
        cid = lax.axis_index("c")
        sid = lax.axis_index("s")
        wid = sid * 2 + cid
        base = wid * PER_W

        pltpu.sync_copy(x_hbm.at[pl.ds(base, PER_W)], idx_v)
        pltpu.sync_copy(pos_hbm, pos_v)

        def gather_copies(c, b):
            off = c * CHUNK
            return (
                pltpu.make_async_copy(
                    tok_hbm.at[idx_v.at[pl.ds(off, 128)]],
                    rows_v.at[b, pl.ds(0, 128)], gsems[b]),
                pltpu.make_async_copy(
                    tok_hbm.at[idx_v.at[pl.ds(off + 128, 72)]],
                    rows_v.at[b, pl.ds(128, 72)], gsems[b]),
            )

        def start_gather(c, b):
            for cp in gather_copies(c, b):
                cp.start()

        def wait_gather(c, b):
            for cp in gather_copies(c, b):
                cp.wait()

        def out_copy(c, b):
            return pltpu.make_async_copy(
                rows_v.at[b], out_hbm.at[pl.ds(base + c * CHUNK, CHUNK)],
                osems[b])

        def add_pos(b):
            def row_body(i, carry):
                for j in range(D // 16):
                    sl = pl.ds(j * 16, 16)
                    rows_v[b, i, sl] = rows_v[b, i, sl] + pos_v[i, sl]
                return carry
            lax.fori_loop(0, CHUNK, row_body, 0)

        start_gather(0, 0)
        start_gather(1, 1)

        def chunk_body(c, b, bnext, guard_first):
            wait_gather(c, b)
            add_pos(b)
            out_copy(c, b).start()
            if guard_first:
                @pl.when(c >= 1)
                def _():
                    out_copy(c - 1, bnext).wait()
            else:
                out_copy(c - 1, bnext).wait()
            return c + 2

        def loop_body(it, carry):
            cbase = it * NBUF
            for j in range(NBUF):
                c = cbase + j
                bnext = (j + 2) % NBUF
                chunk_body(c, j, bnext, guard_first=(j == 0))
                start_gather(c + 2, bnext)
            return carry

        lax.fori_loop(0, (NCHUNK - 2) // NBUF, loop_body, 0)

        chunk_body(NCHUNK - 2, 0, 2, guard_first=False)
        chunk_body(NCHUNK - 1, 1, 0, guard_first=False)
        out_copy(NCHUNK - 1, 1).wait()

    return k


_sc_kernel = _make_sc_kernel()


def kernel(x, token_table, pos_table):
    xf = x.reshape(N).astype(jnp.int32)
    out = _sc_kernel(xf, token_table, pos_table)
    return out.reshape(B, S, D)


# speedup vs baseline: 9.4047x; 1.0417x over previous
import functools

import jax
import jax.numpy as jnp
from jax import lax
from jax.experimental import pallas as pl
from jax.experimental.pallas import tpu as pltpu
from jax.experimental.pallas import tpu_sc as plsc

B = 4096
S = 200
D = 128
N = B * S
NW = 32
BW = B // NW
NCHUNK = S
NBUF = 4
VL = 16
NV = D // VL


def _make_sc_kernel():
    mesh = plsc.VectorSubcoreMesh(core_axis_name="c", subcore_axis_name="s")

    @functools.partial(
        pl.kernel,
        mesh=mesh,
        out_type=jax.ShapeDtypeStruct((N, D), jnp.float32),
        scratch_types=[
            pltpu.VMEM((NCHUNK, BW), jnp.int32),
            pltpu.VMEM((S, D), jnp.float32),
            pltpu.VMEM((BW,), jnp.int32),
            pltpu.VMEM((NBUF, BW, D), jnp.float32),
            pltpu.VMEM((NBUF, BW), jnp.int32),
            pltpu.SemaphoreType.DMA,
            pltpu.SemaphoreType.DMA,
            pltpu.SemaphoreType.DMA,
            pltpu.SemaphoreType.DMA,
            pltpu.SemaphoreType.DMA,
            pltpu.SemaphoreType.DMA,
            pltpu.SemaphoreType.DMA,
            pltpu.SemaphoreType.DMA,
        ],
    )
    def k(xw_hbm, tok_hbm, pos_hbm, brow_hbm, out_hbm,
          idx_v, pos_v, brow_v, buf, sidx,
          g0, g1, g2, g3, s0, s1, s2, s3):
        gsems = (g0, g1, g2, g3)
        ssems = (s0, s1, s2, s3)
        cid = lax.axis_index("c")
        sid = lax.axis_index("s")
        wid = sid * 2 + cid
        out_base = wid * (BW * S)

        pltpu.sync_copy(xw_hbm.at[wid], idx_v)
        pltpu.sync_copy(pos_hbm, pos_v)
        pltpu.sync_copy(brow_hbm, brow_v)

        bvecs = [brow_v[pl.ds(j * VL, VL)] for j in range(NV)]

        def gather_copy(c, slot):
            return pltpu.make_async_copy(
                tok_hbm.at[idx_v.at[c]], buf.at[slot], gsems[slot])

        def scatter_copy(slot):
            return pltpu.make_async_copy(
                buf.at[slot], out_hbm.at[sidx.at[slot]], ssems[slot])

        def chunk_body(c, slot, wait_sc, more):
            if more:
                nslot = (slot + 2) % NBUF
                if wait_sc:
                    scatter_copy(nslot).wait()
                gather_copy(c + 2, nslot).start()

            gather_copy(c, slot).wait()

            pvecs = [pos_v[c, pl.ds(j * VL, VL)] for j in range(NV)]
            row_off = out_base + c
            for j in range(NV):
                sidx[slot, pl.ds(j * VL, VL)] = bvecs[j] + row_off

            def row_body(k2, carry):
                r = k2 * 2
                for dr in (0, 1):
                    for j in range(NV):
                        sl = pl.ds(j * VL, VL)
                        buf[slot, r + dr, sl] = buf[slot, r + dr, sl] + pvecs[j]
                return carry
            lax.fori_loop(0, BW // 2, row_body, 0)

            scatter_copy(slot).start()

        gather_copy(0, 0).start()
        gather_copy(1, 1).start()
        chunk_body(0, 0, wait_sc=False, more=True)
        chunk_body(1, 1, wait_sc=False, more=True)
        chunk_body(2, 2, wait_sc=True, more=True)
        chunk_body(3, 3, wait_sc=True, more=True)

        def loop_body(it, carry):
            cbase = it * NBUF
            for j in range(NBUF):
                chunk_body(cbase + j, j, wait_sc=True, more=True)
            return carry

        lax.fori_loop(1, (NCHUNK - NBUF) // NBUF, loop_body, 0)
        chunk_body(NCHUNK - 4, 0, wait_sc=True, more=True)
        chunk_body(NCHUNK - 3, 1, wait_sc=True, more=True)
        chunk_body(NCHUNK - 2, 2, wait_sc=False, more=False)
        chunk_body(NCHUNK - 1, 3, wait_sc=False, more=False)
        for slot in range(NBUF):
            scatter_copy(slot).wait()

    return k


_sc_kernel = _make_sc_kernel()


def kernel(x, token_table, pos_table):
    xw = x.astype(jnp.int32).reshape(NW, BW, S).transpose(0, 2, 1)
    brow = jnp.arange(BW, dtype=jnp.int32) * S
    out = _sc_kernel(xw, token_table, pos_table, brow)
    return out.reshape(B, S, D)


# speedup vs baseline: 9.4503x; 1.0048x over previous
import functools

import jax
import jax.numpy as jnp
from jax import lax
from jax.experimental import pallas as pl
from jax.experimental.pallas import tpu as pltpu
from jax.experimental.pallas import tpu_sc as plsc

B = 4096
S = 200
D = 128
N = B * S
NW = 32
BW = B // NW
NCHUNK = S
NBUF = 5
VL = 16
NV = D // VL
AUXW = BW + D


def _make_sc_kernel():
    mesh = plsc.VectorSubcoreMesh(core_axis_name="c", subcore_axis_name="s")

    @functools.partial(
        pl.kernel,
        mesh=mesh,
        out_type=jax.ShapeDtypeStruct((N, D), jnp.float32),
        scratch_types=[
            pltpu.VMEM((NBUF, 1, AUXW), jnp.int32),
            pltpu.VMEM((BW,), jnp.int32),
            pltpu.VMEM((NBUF, BW, D), jnp.float32),
            pltpu.VMEM((NBUF, BW), jnp.int32),
        ] + [pltpu.SemaphoreType.DMA] * (3 * NBUF),
    )
    def k(aux_hbm, tok_hbm, brow_hbm, out_hbm,
          auxc, brow_v, buf, sidx, *sems):
        asems = sems[0:NBUF]
        gsems = sems[NBUF:2 * NBUF]
        ssems = sems[2 * NBUF:3 * NBUF]
        cid = lax.axis_index("c")
        sid = lax.axis_index("s")
        wid = sid * 2 + cid
        out_base = wid * (BW * S)

        pltpu.sync_copy(brow_hbm, brow_v)
        bvecs = [brow_v[pl.ds(j * VL, VL)] for j in range(NV)]

        def aux_copy(c, slot):
            return pltpu.make_async_copy(
                aux_hbm.at[wid, pl.ds(c, 1)], auxc.at[slot], asems[slot])

        def gather_copy(c, slot):
            return pltpu.make_async_copy(
                tok_hbm.at[auxc.at[slot, 0, pl.ds(0, BW)]], buf.at[slot],
                gsems[slot])

        def scatter_copy(slot):
            return pltpu.make_async_copy(
                buf.at[slot], out_hbm.at[sidx.at[slot]], ssems[slot])

        def chunk_body(c, slot, wait_sc=True, more_aux=True, more_g=True):
            if more_aux:
                aux_copy(c + 4, (slot + 4) % NBUF).start()
            if more_g:
                nslot = (slot + 3) % NBUF
                if wait_sc:
                    scatter_copy(nslot).wait()
                aux_copy(c + 3, nslot).wait()
                gather_copy(c + 3, nslot).start()

            gather_copy(c, slot).wait()

            pvecs = [
                lax.bitcast_convert_type(
                    auxc[slot, 0, pl.ds(BW + j * VL, VL)], jnp.float32)
                for j in range(NV)
            ]
            row_off = out_base + c
            for j in range(NV):
                sidx[slot, pl.ds(j * VL, VL)] = bvecs[j] + row_off

            @plsc.parallel_loop(0, BW, unroll=4)
            def row_body(r):
                for j in range(NV):
                    sl = pl.ds(j * VL, VL)
                    buf[slot, r, sl] = buf[slot, r, sl] + pvecs[j]

            scatter_copy(slot).start()

        for c in range(4):
            aux_copy(c, c).start()
        for c in range(3):
            aux_copy(c, c).wait()
            gather_copy(c, c).start()

        chunk_body(0, 0, wait_sc=False)
        chunk_body(1, 1, wait_sc=False)
        chunk_body(2, 2)
        chunk_body(3, 3)
        chunk_body(4, 4)

        def loop_body(it, carry):
            cbase = it * NBUF
            for j in range(NBUF):
                chunk_body(cbase + j, j)
            return carry

        lax.fori_loop(1, NCHUNK // NBUF - 1, loop_body, 0)
        chunk_body(NCHUNK - 5, 0)
        chunk_body(NCHUNK - 4, 1, more_aux=False)
        chunk_body(NCHUNK - 3, 2, more_aux=False, more_g=False)
        chunk_body(NCHUNK - 2, 3, more_aux=False, more_g=False)
        chunk_body(NCHUNK - 1, 4, more_aux=False, more_g=False)
        for slot in range(NBUF):
            scatter_copy(slot).wait()

    return k


_sc_kernel = _make_sc_kernel()


def kernel(x, token_table, pos_table):
    xw = x.astype(jnp.int32).reshape(NW, BW, S).transpose(0, 2, 1)
    pos_i = lax.bitcast_convert_type(pos_table, jnp.int32)
    aux = jnp.concatenate(
        [xw, jnp.broadcast_to(pos_i[None], (NW, S, D))], axis=2)
    brow = jnp.arange(BW, dtype=jnp.int32) * S
    out = _sc_kernel(aux, token_table, brow)
    return out.reshape(B, S, D)


# speedup vs baseline: 9.4674x; 1.0018x over previous
import functools

import jax
import jax.numpy as jnp
from jax import lax
from jax.experimental import pallas as pl
from jax.experimental.pallas import tpu as pltpu
from jax.experimental.pallas import tpu_sc as plsc

B = 4096
S = 200
D = 128
N = B * S
NW = 32
BW = B // NW
NCHUNK = S
NBUF = 5
VL = 16
NV = D // VL
AUXW = BW + D


def _make_sc_kernel():
    mesh = plsc.VectorSubcoreMesh(core_axis_name="c", subcore_axis_name="s")

    @functools.partial(
        pl.kernel,
        mesh=mesh,
        out_type=jax.ShapeDtypeStruct((B, S, D), jnp.float32),
        scratch_types=[
            pltpu.VMEM((NBUF, 1, AUXW), jnp.int32),
            pltpu.VMEM((NBUF, BW, D), jnp.float32),
        ] + [pltpu.SemaphoreType.DMA] * (3 * NBUF),
    )
    def k(aux_hbm, tok_hbm, out_hbm, auxc, buf, *sems):
        asems = sems[0:NBUF]
        gsems = sems[NBUF:2 * NBUF]
        ssems = sems[2 * NBUF:3 * NBUF]
        cid = lax.axis_index("c")
        sid = lax.axis_index("s")
        wid = sid * 2 + cid
        seq_base = wid * BW

        def aux_copy(c, slot):
            return pltpu.make_async_copy(
                aux_hbm.at[wid, pl.ds(c, 1)], auxc.at[slot], asems[slot])

        def gather_copy(c, slot):
            return pltpu.make_async_copy(
                tok_hbm.at[auxc.at[slot, 0, pl.ds(0, BW)]], buf.at[slot],
                gsems[slot])

        def scatter_copy(c, slot):
            return pltpu.make_async_copy(
                buf.at[slot].reshape(BW, 1, D),
                out_hbm.at[pl.ds(seq_base, BW), pl.ds(c, 1)],
                ssems[slot])

        def chunk_body(c, slot, wait_sc=True, more_aux=True, more_g=True):
            if more_aux:
                aux_copy(c + 4, (slot + 4) % NBUF).start()
            if more_g:
                nslot = (slot + 3) % NBUF
                if wait_sc:
                    scatter_copy(c - 2, nslot).wait()
                aux_copy(c + 3, nslot).wait()
                gather_copy(c + 3, nslot).start()

            gather_copy(c, slot).wait()

            pvecs = [
                lax.bitcast_convert_type(
                    auxc[slot, 0, pl.ds(BW + j * VL, VL)], jnp.float32)
                for j in range(NV)
            ]
            @plsc.parallel_loop(0, BW, unroll=4)
            def row_body(r):
                for j in range(NV):
                    sl = pl.ds(j * VL, VL)
                    buf[slot, r, sl] = buf[slot, r, sl] + pvecs[j]

            scatter_copy(c, slot).start()

        for c in range(4):
            aux_copy(c, c).start()
        for c in range(3):
            aux_copy(c, c).wait()
            gather_copy(c, c).start()

        chunk_body(0, 0, wait_sc=False)
        chunk_body(1, 1, wait_sc=False)
        chunk_body(2, 2)
        chunk_body(3, 3)
        chunk_body(4, 4)

        def loop_body(it, carry):
            cbase = it * NBUF
            for j in range(NBUF):
                chunk_body(cbase + j, j)
            return carry

        lax.fori_loop(1, NCHUNK // NBUF - 1, loop_body, 0)
        chunk_body(NCHUNK - 5, 0)
        chunk_body(NCHUNK - 4, 1, more_aux=False)
        chunk_body(NCHUNK - 3, 2, more_aux=False, more_g=False)
        chunk_body(NCHUNK - 2, 3, more_aux=False, more_g=False)
        chunk_body(NCHUNK - 1, 4, more_aux=False, more_g=False)
        for slot in range(NBUF):
            scatter_copy(NCHUNK - NBUF + slot, slot).wait()

    return k


_sc_kernel = _make_sc_kernel()


def kernel(x, token_table, pos_table):
    xw = x.astype(jnp.int32).reshape(NW, BW, S).transpose(0, 2, 1)
    pos_i = lax.bitcast_convert_type(pos_table, jnp.int32)
    aux = jnp.concatenate(
        [xw, jnp.broadcast_to(pos_i[None], (NW, S, D))], axis=2)
    return _sc_kernel(aux, token_table)
